# Initial kernel scaffold; baseline (speedup 1.0000x reference)
import jax
import jax.numpy as jnp
from jax.experimental import pallas as pl


def kernel(iword, owords, table_i, table_o):
    raise NotImplementedError("write your pallas kernel here")


# speedup vs baseline: 3.3084x; 3.3084x over previous
import functools

import jax
import jax.numpy as jnp
from jax import lax
from jax.experimental import pallas as pl
from jax.experimental.pallas import tpu as pltpu
from jax.experimental.pallas import tpu_sc as plsc

D = 32
N_NEGS = 20
GROUP = 128
G = 12


def _sc_gather(table_i, table_o, iword_i32, idx_o_flat):
    B = iword_i32.shape[0]
    R = idx_o_flat.shape[0]
    info = plsc.get_sparse_core_info()
    NC, NS = info.num_cores, info.num_subcores
    NW = NC * NS
    rows_w = R // NW
    STEP = G * GROUP
    n_steps = rows_w // STEP
    assert rows_w % STEP == 0
    b_w = B // NW

    mesh = plsc.VectorSubcoreMesh(core_axis_name="c", subcore_axis_name="s")

    @functools.partial(
        pl.kernel, mesh=mesh,
        compiler_params=pltpu.CompilerParams(use_tc_tiling_on_sc=False),
        out_type=(
            jax.ShapeDtypeStruct((B, D), jnp.float32),
            jax.ShapeDtypeStruct((R, D), jnp.float32),
        ),
        scratch_types=[
            pltpu.VMEM((b_w,), jnp.int32),
            pltpu.VMEM((b_w, D), jnp.float32),
            pltpu.VMEM((STEP,), jnp.int32),
            pltpu.VMEM((STEP, D), jnp.float32),
            pltpu.SemaphoreType.DMA,
        ],
    )
    def gather_kernel(ti_hbm, to_hbm, iw_hbm, io_hbm, iv_out, ov_out,
                      iw_v, iv_v, idx_v, rows_v, sem):
        wid = lax.axis_index("s") * NC + lax.axis_index("c")

        ib = wid * b_w
        pltpu.sync_copy(iw_hbm.at[pl.ds(ib, b_w)], iw_v)
        pltpu.async_copy(ti_hbm.at[iw_v], iv_v, sem).wait()
        pltpu.sync_copy(iv_v, iv_out.at[pl.ds(ib, b_w)])

        def step(t, carry):
            base = wid * rows_w + t * STEP
            pltpu.sync_copy(io_hbm.at[pl.ds(base, STEP)], idx_v)
            copies = [
                pltpu.async_copy(to_hbm.at[idx_v.at[pl.ds(j * GROUP, GROUP)]],
                                 rows_v.at[pl.ds(j * GROUP, GROUP)], sem)
                for j in range(G)
            ]
            for c in copies:
                c.wait()
            pltpu.sync_copy(rows_v, ov_out.at[pl.ds(base, STEP)])
            return carry

        lax.fori_loop(0, n_steps, step, 0)

    return gather_kernel(table_i, table_o, iword_i32, idx_o_flat)


def _tc_loss_sum(iv, ov3, C):
    B, RPB, _ = ov3.shape
    BB = 64
    grid = B // BB

    def body(iv_ref, ov_ref, out_ref):
        i = pl.program_id(0)
        ivb = iv_ref[...]
        ovb = ov_ref[...]
        d = jnp.sum(ovb * ivb[:, None, :], axis=2)
        col = lax.broadcasted_iota(jnp.int32, (BB, RPB), 1)
        x = jnp.where(col < C, d, -d)
        ls = jnp.minimum(x, 0.0) - jnp.log(1.0 + jnp.exp(-jnp.abs(x)))
        part = jnp.sum(ls)

        @pl.when(i == 0)
        def _():
            out_ref[...] = jnp.zeros_like(out_ref)

        out_ref[...] += jnp.full((1, 1), part, jnp.float32)

    out = pl.pallas_call(
        body,
        grid=(grid,),
        in_specs=[
            pl.BlockSpec((BB, D), lambda i: (i, 0)),
            pl.BlockSpec((BB, RPB, D), lambda i: (i, 0, 0)),
        ],
        out_specs=pl.BlockSpec((1, 1), lambda i: (0, 0)),
        out_shape=jax.ShapeDtypeStruct((1, 1), jnp.float32),
    )(iv, ov3)
    return out[0, 0]


def kernel(iword, owords, table_i, table_o):
    B = iword.shape[0]
    C = owords.shape[1]
    V = table_i.shape[0]
    RPB = C * (1 + N_NEGS)

    nwords = jax.random.randint(jax.random.key(1), (B, C * N_NEGS), 0, V - 1)

    idx_o = jnp.concatenate(
        [owords.astype(jnp.int32), nwords.astype(jnp.int32)], axis=1
    ).reshape(B * RPB)

    iv, ov = _sc_gather(table_i, table_o, iword.astype(jnp.int32), idx_o)
    total = _tc_loss_sum(iv, ov.reshape(B, RPB, D), C)
    return -total / jnp.float32(B * C)


# speedup vs baseline: 3.8351x; 1.1592x over previous
import functools

import jax
import jax.numpy as jnp
from jax import lax
from jax.experimental import pallas as pl
from jax.experimental.pallas import tpu as pltpu
from jax.experimental.pallas import tpu_sc as plsc

D = 32
N_NEGS = 20
RPB = 420
RPB_PAD = 432
GPB = RPB_PAD // 16
BPC = 2
CH_ROWS = BPC * RPB_PAD
GSIZES = (128, 128, 128, 128, 128, 128, 96)
assert sum(GSIZES) == CH_ROWS


def _sc_dots(table_i, table_o, iword_i32, idx_pad_flat):
    B = iword_i32.shape[0]
    R2 = idx_pad_flat.shape[0]
    info = plsc.get_sparse_core_info()
    NC, NS = info.num_cores, info.num_subcores
    NW = NC * NS
    b_w = B // NW
    rows_w = R2 // NW
    n_chunks = rows_w // CH_ROWS
    assert rows_w % CH_ROWS == 0 and n_chunks % 2 == 0 and b_w % BPC == 0

    mesh = plsc.VectorSubcoreMesh(core_axis_name="c", subcore_axis_name="s")

    @functools.partial(
        pl.kernel, mesh=mesh,
        compiler_params=pltpu.CompilerParams(
            use_tc_tiling_on_sc=False, needs_layout_passes=False),
        out_type=jax.ShapeDtypeStruct((R2,), jnp.float32),
        scratch_types=[
            pltpu.VMEM((b_w,), jnp.int32),
            pltpu.VMEM((b_w, D), jnp.float32),
            pltpu.VMEM((rows_w,), jnp.int32),
            pltpu.VMEM((CH_ROWS, D), jnp.float32),
            pltpu.VMEM((CH_ROWS, D), jnp.float32),
            pltpu.VMEM((CH_ROWS,), jnp.float32),
            pltpu.VMEM((CH_ROWS,), jnp.float32),
            pltpu.SemaphoreType.DMA,
            pltpu.SemaphoreType.DMA,
        ],
    )
    def dots_kernel(ti_hbm, to_hbm, iw_hbm, io_hbm, dots_out,
                    iw_v, iv_v, idx_v, rows_a, rows_b, dots_a, dots_b,
                    sem_a, sem_b):
        wid = lax.axis_index("s") * NC + lax.axis_index("c")
        base_w = wid * rows_w

        pltpu.sync_copy(iw_hbm.at[pl.ds(wid * b_w, b_w)], iw_v)
        pltpu.make_async_copy(ti_hbm.at[iw_v], iv_v, sem_a).start()
        pltpu.sync_copy(io_hbm.at[pl.ds(base_w, rows_w)], idx_v)
        pltpu.make_async_copy(ti_hbm.at[iw_v], iv_v, sem_a).wait()

        iota16 = lax.iota(jnp.int32, 16)

        def fire(c, rows_v, sem):
            o = 0
            for sz in GSIZES:
                pltpu.make_async_copy(
                    to_hbm.at[idx_v.at[pl.ds(c * CH_ROWS + o, sz)]],
                    rows_v.at[pl.ds(o, sz)], sem).start()
                o += sz

        def drain(rows_v, sem):
            o = 0
            for sz in GSIZES:
                pltpu.make_async_copy(
                    to_hbm.at[idx_v.at[pl.ds(o, sz)]],
                    rows_v.at[pl.ds(o, sz)], sem).wait()
                o += sz

        def process(c, rows_v, dots_v):
            b0 = c * BPC

            def grp(g, carry):
                bl = b0 + g // GPB
                rowv = iota16 + g * 16
                iv_lo = iv_v[bl, pl.ds(0, 16)]
                iv_hi = iv_v[bl, pl.ds(16, 16)]
                acc = jnp.zeros((16,), jnp.float32)
                for k in range(D):
                    colv = jnp.full((16,), k, jnp.int32)
                    cvec = plsc.load_gather(rows_v, [rowv, colv])
                    s = iv_lo[k] if k < 16 else iv_hi[k - 16]
                    acc = acc + cvec * s
                dots_v[pl.ds(g * 16, 16)] = acc
                return carry

            lax.fori_loop(0, BPC * GPB, grp, 0)
            pltpu.sync_copy(dots_v,
                            dots_out.at[pl.ds(base_w + c * CH_ROWS, CH_ROWS)])

        fire(0, rows_a, sem_a)

        def loop(t, carry):
            ca = 2 * t
            fire(ca + 1, rows_b, sem_b)
            drain(rows_a, sem_a)
            process(ca, rows_a, dots_a)
            fire(lax.rem(ca + 2, n_chunks), rows_a, sem_a)
            drain(rows_b, sem_b)
            process(ca + 1, rows_b, dots_b)
            return carry

        lax.fori_loop(0, n_chunks // 2, loop, 0)
        drain(rows_a, sem_a)

    return dots_kernel(table_i, table_o, iword_i32, idx_pad_flat)


def _tc_loss_sum(dots2d, C):
    B, _ = dots2d.shape

    def body(d_ref, out_ref):
        d = d_ref[...]
        col = lax.broadcasted_iota(jnp.int32, (B, RPB_PAD), 1)
        x = jnp.where(col < C, d, -d)
        ls = jnp.minimum(x, 0.0) - jnp.log(1.0 + jnp.exp(-jnp.abs(x)))
        out_ref[...] = jnp.full(
            (1, 1), jnp.sum(jnp.where(col < RPB, ls, 0.0)), jnp.float32)

    out = pl.pallas_call(
        body,
        out_shape=jax.ShapeDtypeStruct((1, 1), jnp.float32),
    )(dots2d)
    return out[0, 0]


def kernel(iword, owords, table_i, table_o):
    B = iword.shape[0]
    C = owords.shape[1]
    V = table_i.shape[0]

    nwords = jax.random.randint(jax.random.key(1), (B, C * N_NEGS), 0, V - 1)

    idx_pad = jnp.concatenate(
        [owords.astype(jnp.int32), nwords.astype(jnp.int32),
         jnp.zeros((B, RPB_PAD - RPB), jnp.int32)], axis=1
    ).reshape(B * RPB_PAD)

    dots = _sc_dots(table_i, table_o, iword.astype(jnp.int32), idx_pad)
    total = _tc_loss_sum(dots.reshape(B, RPB_PAD), C)
    return -total / jnp.float32(B * C)
